# Initial kernel scaffold; baseline (speedup 1.0000x reference)
import jax
import jax.numpy as jnp
from jax.experimental import pallas as pl


def kernel(x, edge_index, W1l, W1r, b1, W2l, W2r, b2, W3l, W3r, b3, W4, b4):
    raise NotImplementedError("write your pallas kernel here")


# speedup vs baseline: 8.2639x; 8.2639x over previous
import jax
import jax.numpy as jnp
from jax import lax
from jax.experimental import pallas as pl
from jax.experimental.pallas import tpu as pltpu
from jax.experimental.pallas import tpu_sc as plsc

N_NODES = 10000
D = 128
NC = 2
NS = 16
NW = NC * NS
CHUNK = 40
GRP = 5
NGRP = 50
EPT = GRP * CHUNK * NGRP
RPT = 640
NACC = NS * RPT


def _sc_agg():
  mesh = plsc.VectorSubcoreMesh(core_axis_name="c", subcore_axis_name="s")
  scratch = [
      pltpu.VMEM((GRP, CHUNK), jnp.int32),
      pltpu.VMEM((GRP, CHUNK), jnp.int32),
      pltpu.VMEM((GRP, CHUNK, D), jnp.float32),
      pltpu.VMEM_SHARED((NACC, D), jnp.float32),
      pltpu.SemaphoreType.DMA,
  ]

  def body(h_hbm, src_hbm, dst_hbm, out_hbm, srcv, dstv, gbuf, acc, gsem):
    c = lax.axis_index("c")
    s = lax.axis_index("s")
    wid = c * NS + s

    zero16 = jnp.zeros((16,), jnp.float32)

    def fill_z(i, carry):
      for j in range(D // 16):
        gbuf[0, i, pl.ds(j * 16, 16)] = zero16
      return carry

    lax.fori_loop(0, CHUNK, fill_z, 0)

    base = s * RPT

    def zero_acc(k, carry):
      off = base + k * CHUNK
      pltpu.sync_copy(gbuf.at[0], acc.at[pl.ds(off, CHUNK)])
      return carry

    lax.fori_loop(0, RPT // CHUNK, zero_acc, 0)

    plsc.subcore_barrier()

    def group(g, carry):
      pltpu.sync_copy(src_hbm.at[wid, g], srcv)
      pltpu.sync_copy(dst_hbm.at[wid, g], dstv)
      descs = []
      for b in range(GRP):
        descs.append(
            pltpu.async_copy(h_hbm.at[srcv.at[b]], gbuf.at[b], gsem))
      for b in range(GRP):
        descs[b].wait()
        pltpu.sync_copy(gbuf.at[b], acc.at[dstv.at[b]], add=True)
      return carry

    lax.fori_loop(0, NGRP, group, 0)

    plsc.subcore_barrier()

    def flush(k, carry):
      off = base + k * CHUNK
      pltpu.sync_copy(acc.at[pl.ds(off, CHUNK)], gbuf.at[0])
      pltpu.sync_copy(gbuf.at[0], out_hbm.at[c, pl.ds(off, CHUNK)])
      return carry

    lax.fori_loop(0, RPT // CHUNK, flush, 0)

  return pl.kernel(
      body,
      out_type=jax.ShapeDtypeStruct((NC, NACC, D), jnp.float32),
      mesh=mesh,
      scratch_types=scratch,
  )


def _sc_counts():
  mesh = plsc.VectorSubcoreMesh(core_axis_name="c", subcore_axis_name="s")
  scratch = [
      pltpu.VMEM((EPT,), jnp.int32),
      pltpu.VMEM((NACC,), jnp.float32),
  ]

  def body(dst_hbm, cnt_hbm, dstv, cntv):
    c = lax.axis_index("c")
    s = lax.axis_index("s")
    wid = c * NS + s

    pltpu.sync_copy(dst_hbm.at[wid], dstv)

    zero16 = jnp.zeros((16,), jnp.float32)
    one16 = jnp.ones((16,), jnp.float32)

    def fill_z(i, carry):
      cntv[pl.ds(i * 16, 16)] = zero16
      return carry

    lax.fori_loop(0, NACC // 16, fill_z, 0)

    def count(e, carry):
      idx = dstv[pl.ds(e * 16, 16)]
      plsc.addupdate_scatter(cntv, [idx], one16)
      return carry

    lax.fori_loop(0, EPT // 16, count, 0)

    pltpu.sync_copy(cntv, cnt_hbm.at[wid])

  return pl.kernel(
      body,
      out_type=jax.ShapeDtypeStruct((NW, NACC), jnp.float32),
      mesh=mesh,
      scratch_types=scratch,
      compiler_params=pltpu.CompilerParams(needs_layout_passes=False),
  )


def _tc_layer(p0, p1, cnt_t, h, Wl, Wr, b, W4=None, b4=None):
  R = 1000
  fused_final = W4 is not None

  def tcb(p0_ref, p1_ref, c_ref, h_ref, wl_ref, wr_ref, b_ref, *rest):
    if fused_final:
      w4_ref, b4_ref, o_ref = rest
    else:
      (o_ref,) = rest
    cnt = jnp.sum(c_ref[:, :], axis=1, keepdims=True)
    mean = (p0_ref[:, :] + p1_ref[:, :]) / jnp.maximum(cnt, 1.0)
    acc = jnp.dot(mean, wl_ref[:, :], preferred_element_type=jnp.float32)
    acc = acc + jnp.dot(h_ref[:, :], wr_ref[:, :],
                        preferred_element_type=jnp.float32)
    acc = acc + b_ref[:, :]
    acc = jnp.maximum(acc, 0.0)
    if fused_final:
      acc = jnp.dot(acc, w4_ref[:, :],
                    preferred_element_type=jnp.float32) + b4_ref[:, :]
    o_ref[:, :] = acc

  row_spec = pl.BlockSpec((R, D), lambda i: (i, 0))
  cnt_spec = pl.BlockSpec((R, NW), lambda i: (i, 0))
  w_spec = pl.BlockSpec((D, D), lambda i: (0, 0))
  b_spec = pl.BlockSpec((1, D), lambda i: (0, 0))
  in_specs = [row_spec, row_spec, cnt_spec, row_spec,
              w_spec, w_spec, b_spec]
  args = [p0, p1, cnt_t, h, Wl, Wr, b.reshape(1, D)]
  if fused_final:
    in_specs += [w_spec, b_spec]
    args += [W4, b4.reshape(1, D)]
  return pl.pallas_call(
      tcb,
      grid=(N_NODES // R,),
      in_specs=in_specs,
      out_specs=row_spec,
      out_shape=jax.ShapeDtypeStruct((N_NODES, D), jnp.float32),
  )(*args)


def kernel(x, edge_index, W1l, W1r, b1, W2l, W2r, b2, W3l, W3r, b3, W4, b4):
  ei = edge_index.astype(jnp.int32)
  src = ei[0].reshape(NW, NGRP, GRP, CHUNK)
  dst = ei[1].reshape(NW, NGRP, GRP, CHUNK)
  dstf = ei[1].reshape(NW, EPT)

  agg = _sc_agg()
  cnt = _sc_counts()(dstf)
  cnt_t = cnt.T[:N_NODES]

  p = agg(x, src, dst)
  h1 = _tc_layer(p[0, :N_NODES], p[1, :N_NODES], cnt_t, x, W1l, W1r, b1)
  p2 = agg(h1, src, dst)
  h2 = _tc_layer(p2[0, :N_NODES], p2[1, :N_NODES], cnt_t, h1, W2l, W2r, b2)
  p3 = agg(h2, src, dst)
  return _tc_layer(p3[0, :N_NODES], p3[1, :N_NODES], cnt_t, h2,
                   W3l, W3r, b3, W4, b4)


# speedup vs baseline: 8.5941x; 1.0400x over previous
import jax
import jax.numpy as jnp
from jax import lax
from jax.experimental import pallas as pl
from jax.experimental.pallas import tpu as pltpu
from jax.experimental.pallas import tpu_sc as plsc

N_NODES = 10000
D = 128
NC = 2
NS = 16
NW = NC * NS
CHUNK = 40
GRP = 5
NGRP = 50
EPT = GRP * CHUNK * NGRP
RPT = 640
NACC = NS * RPT


def _sc_agg():
  mesh = plsc.VectorSubcoreMesh(core_axis_name="c", subcore_axis_name="s")
  scratch = [
      pltpu.VMEM((GRP, CHUNK), jnp.int32),
      pltpu.VMEM((GRP, CHUNK), jnp.int32),
      pltpu.VMEM((GRP, CHUNK, D), jnp.float32),
      pltpu.VMEM_SHARED((NACC, D), jnp.float32),
      pltpu.SemaphoreType.DMA,
      pltpu.SemaphoreType.DMA,
  ]

  def body(h_hbm, src_hbm, dst_hbm, out_hbm, srcv, dstv, gbuf, acc, gsem,
           ssem):
    c = lax.axis_index("c")
    s = lax.axis_index("s")
    wid = c * NS + s

    zero16 = jnp.zeros((16,), jnp.float32)

    def fill_z(i, carry):
      for j in range(D // 16):
        gbuf[0, i, pl.ds(j * 16, 16)] = zero16
      return carry

    lax.fori_loop(0, CHUNK, fill_z, 0)

    base = s * RPT

    def zero_acc(k, carry):
      off = base + k * CHUNK
      pltpu.sync_copy(gbuf.at[0], acc.at[pl.ds(off, CHUNK)])
      return carry

    lax.fori_loop(0, RPT // CHUNK, zero_acc, 0)

    plsc.subcore_barrier()

    def group(g, carry):
      pltpu.sync_copy(src_hbm.at[wid, g], srcv)
      pltpu.sync_copy(dst_hbm.at[wid, g], dstv)
      descs = []
      for b in range(GRP):
        descs.append(
            pltpu.async_copy(h_hbm.at[srcv.at[b]], gbuf.at[b], gsem))
      sdescs = []
      for b in range(GRP):
        descs[b].wait()
        sdescs.append(
            pltpu.async_copy(gbuf.at[b], acc.at[dstv.at[b]], ssem, add=True))
      for b in range(GRP):
        sdescs[b].wait()
      return carry

    lax.fori_loop(0, NGRP, group, 0)

    plsc.subcore_barrier()

    def flush(k, carry):
      off = base + k * CHUNK
      pltpu.sync_copy(acc.at[pl.ds(off, CHUNK)], gbuf.at[0])
      pltpu.sync_copy(gbuf.at[0], out_hbm.at[c, pl.ds(off, CHUNK)])
      return carry

    lax.fori_loop(0, RPT // CHUNK, flush, 0)

  return pl.kernel(
      body,
      out_type=jax.ShapeDtypeStruct((NC, NACC, D), jnp.float32),
      mesh=mesh,
      scratch_types=scratch,
  )


def _sc_counts():
  mesh = plsc.VectorSubcoreMesh(core_axis_name="c", subcore_axis_name="s")
  scratch = [
      pltpu.VMEM((EPT,), jnp.int32),
      pltpu.VMEM((NACC,), jnp.float32),
  ]

  def body(dst_hbm, cnt_hbm, dstv, cntv):
    c = lax.axis_index("c")
    s = lax.axis_index("s")
    wid = c * NS + s

    pltpu.sync_copy(dst_hbm.at[wid], dstv)

    zero16 = jnp.zeros((16,), jnp.float32)
    one16 = jnp.ones((16,), jnp.float32)

    def fill_z(i, carry):
      cntv[pl.ds(i * 16, 16)] = zero16
      return carry

    lax.fori_loop(0, NACC // 16, fill_z, 0)

    def count(e, carry):
      idx = dstv[pl.ds(e * 16, 16)]
      plsc.addupdate_scatter(cntv, [idx], one16)
      return carry

    lax.fori_loop(0, EPT // 16, count, 0)

    pltpu.sync_copy(cntv, cnt_hbm.at[wid])

  return pl.kernel(
      body,
      out_type=jax.ShapeDtypeStruct((NW, NACC), jnp.float32),
      mesh=mesh,
      scratch_types=scratch,
      compiler_params=pltpu.CompilerParams(needs_layout_passes=False),
  )


def _tc_layer(p0, p1, cnt_t, h, Wl, Wr, b, W4=None, b4=None):
  R = 1000
  fused_final = W4 is not None

  def tcb(p0_ref, p1_ref, c_ref, h_ref, wl_ref, wr_ref, b_ref, *rest):
    if fused_final:
      w4_ref, b4_ref, o_ref = rest
    else:
      (o_ref,) = rest
    cnt = jnp.sum(c_ref[:, :], axis=1, keepdims=True)
    mean = (p0_ref[:, :] + p1_ref[:, :]) / jnp.maximum(cnt, 1.0)
    acc = jnp.dot(mean, wl_ref[:, :], preferred_element_type=jnp.float32)
    acc = acc + jnp.dot(h_ref[:, :], wr_ref[:, :],
                        preferred_element_type=jnp.float32)
    acc = acc + b_ref[:, :]
    acc = jnp.maximum(acc, 0.0)
    if fused_final:
      acc = jnp.dot(acc, w4_ref[:, :],
                    preferred_element_type=jnp.float32) + b4_ref[:, :]
    o_ref[:, :] = acc

  row_spec = pl.BlockSpec((R, D), lambda i: (i, 0))
  cnt_spec = pl.BlockSpec((R, NW), lambda i: (i, 0))
  w_spec = pl.BlockSpec((D, D), lambda i: (0, 0))
  b_spec = pl.BlockSpec((1, D), lambda i: (0, 0))
  in_specs = [row_spec, row_spec, cnt_spec, row_spec,
              w_spec, w_spec, b_spec]
  args = [p0, p1, cnt_t, h, Wl, Wr, b.reshape(1, D)]
  if fused_final:
    in_specs += [w_spec, b_spec]
    args += [W4, b4.reshape(1, D)]
  return pl.pallas_call(
      tcb,
      grid=(N_NODES // R,),
      in_specs=in_specs,
      out_specs=row_spec,
      out_shape=jax.ShapeDtypeStruct((N_NODES, D), jnp.float32),
  )(*args)


def kernel(x, edge_index, W1l, W1r, b1, W2l, W2r, b2, W3l, W3r, b3, W4, b4):
  ei = edge_index.astype(jnp.int32)
  src = ei[0].reshape(NW, NGRP, GRP, CHUNK)
  dst = ei[1].reshape(NW, NGRP, GRP, CHUNK)
  dstf = ei[1].reshape(NW, EPT)

  agg = _sc_agg()
  cnt = _sc_counts()(dstf)
  cnt_t = cnt.T[:N_NODES]

  p = agg(x, src, dst)
  h1 = _tc_layer(p[0, :N_NODES], p[1, :N_NODES], cnt_t, x, W1l, W1r, b1)
  p2 = agg(h1, src, dst)
  h2 = _tc_layer(p2[0, :N_NODES], p2[1, :N_NODES], cnt_t, h1, W2l, W2r, b2)
  p3 = agg(h2, src, dst)
  return _tc_layer(p3[0, :N_NODES], p3[1, :N_NODES], cnt_t, h2,
                   W3l, W3r, b3, W4, b4)


# speedup vs baseline: 9.2611x; 1.0776x over previous
import jax
import jax.numpy as jnp
from jax import lax
from jax.experimental import pallas as pl
from jax.experimental.pallas import tpu as pltpu
from jax.experimental.pallas import tpu_sc as plsc

N_NODES = 10000
D = 128
NC = 2
NS = 16
NW = NC * NS
CHUNK = 40
GRP = 5
NGRP = 50
EPT = GRP * CHUNK * NGRP
RPT = 640
NACC = NS * RPT


def _sc_agg():
  mesh = plsc.VectorSubcoreMesh(core_axis_name="c", subcore_axis_name="s")
  scratch = [
      pltpu.VMEM((GRP, CHUNK), jnp.int32),
      pltpu.VMEM((GRP, CHUNK), jnp.int32),
      pltpu.VMEM((GRP, CHUNK), jnp.int32),
      pltpu.VMEM((GRP, CHUNK, D), jnp.float32),
      pltpu.VMEM_SHARED((NACC, D), jnp.float32),
      pltpu.SemaphoreType.DMA,
      pltpu.SemaphoreType.DMA,
  ]

  def body(h_hbm, src_hbm, dst_hbm, out_hbm, srcv, dstva, dstvb, gbuf, acc,
           gsem, ssem):
    c = lax.axis_index("c")
    s = lax.axis_index("s")
    wid = c * NS + s

    zero16 = jnp.zeros((16,), jnp.float32)

    def fill_z(i, carry):
      for j in range(D // 16):
        gbuf[0, i, pl.ds(j * 16, 16)] = zero16
      return carry

    lax.fori_loop(0, CHUNK, fill_z, 0)

    base = s * RPT

    def zero_acc(k, carry):
      off = base + k * CHUNK
      pltpu.sync_copy(gbuf.at[0], acc.at[pl.ds(off, CHUNK)])
      return carry

    lax.fori_loop(0, RPT // CHUNK, zero_acc, 0)

    plsc.subcore_barrier()

    def half(g, dstv, first):
      pltpu.sync_copy(src_hbm.at[wid, g], srcv)
      pltpu.sync_copy(dst_hbm.at[wid, g], dstv)
      descs = []
      for b in range(GRP):
        if first is None:
          pltpu.make_async_copy(gbuf.at[b], acc.at[dstv.at[b]], ssem).wait()
        else:
          @pl.when(first)
          def _drain():
            pltpu.make_async_copy(gbuf.at[b], acc.at[dstv.at[b]],
                                  ssem).wait()
        descs.append(
            pltpu.async_copy(h_hbm.at[srcv.at[b]], gbuf.at[b], gsem))
      for b in range(GRP):
        descs[b].wait()
        pltpu.async_copy(gbuf.at[b], acc.at[dstv.at[b]], ssem, add=True)

    def group_pair(gg, carry):
      half(2 * gg, dstva, gg > 0)
      half(2 * gg + 1, dstvb, None)
      return carry

    lax.fori_loop(0, NGRP // 2, group_pair, 0)

    for b in range(GRP):
      pltpu.make_async_copy(gbuf.at[b], acc.at[dstvb.at[b]], ssem).wait()

    plsc.subcore_barrier()

    def flush(k, carry):
      off = base + k * CHUNK
      pltpu.sync_copy(acc.at[pl.ds(off, CHUNK)], gbuf.at[0])
      pltpu.sync_copy(gbuf.at[0], out_hbm.at[c, pl.ds(off, CHUNK)])
      return carry

    lax.fori_loop(0, RPT // CHUNK, flush, 0)

  return pl.kernel(
      body,
      out_type=jax.ShapeDtypeStruct((NC, NACC, D), jnp.float32),
      mesh=mesh,
      scratch_types=scratch,
  )


def _sc_counts():
  mesh = plsc.VectorSubcoreMesh(core_axis_name="c", subcore_axis_name="s")
  scratch = [
      pltpu.VMEM((EPT,), jnp.int32),
      pltpu.VMEM((NACC,), jnp.float32),
  ]

  def body(dst_hbm, cnt_hbm, dstv, cntv):
    c = lax.axis_index("c")
    s = lax.axis_index("s")
    wid = c * NS + s

    pltpu.sync_copy(dst_hbm.at[wid], dstv)

    zero16 = jnp.zeros((16,), jnp.float32)
    one16 = jnp.ones((16,), jnp.float32)

    def fill_z(i, carry):
      cntv[pl.ds(i * 16, 16)] = zero16
      return carry

    lax.fori_loop(0, NACC // 16, fill_z, 0)

    def count(e, carry):
      idx = dstv[pl.ds(e * 16, 16)]
      plsc.addupdate_scatter(cntv, [idx], one16)
      return carry

    lax.fori_loop(0, EPT // 16, count, 0)

    pltpu.sync_copy(cntv, cnt_hbm.at[wid])

  return pl.kernel(
      body,
      out_type=jax.ShapeDtypeStruct((NW, NACC), jnp.float32),
      mesh=mesh,
      scratch_types=scratch,
      compiler_params=pltpu.CompilerParams(needs_layout_passes=False),
  )


def _tc_layer(p0, p1, cnt_t, h, Wl, Wr, b, W4=None, b4=None):
  R = 1000
  fused_final = W4 is not None

  def tcb(p0_ref, p1_ref, c_ref, h_ref, wl_ref, wr_ref, b_ref, *rest):
    if fused_final:
      w4_ref, b4_ref, o_ref = rest
    else:
      (o_ref,) = rest
    cnt = jnp.sum(c_ref[:, :], axis=1, keepdims=True)
    mean = (p0_ref[:, :] + p1_ref[:, :]) / jnp.maximum(cnt, 1.0)
    acc = jnp.dot(mean, wl_ref[:, :], preferred_element_type=jnp.float32)
    acc = acc + jnp.dot(h_ref[:, :], wr_ref[:, :],
                        preferred_element_type=jnp.float32)
    acc = acc + b_ref[:, :]
    acc = jnp.maximum(acc, 0.0)
    if fused_final:
      acc = jnp.dot(acc, w4_ref[:, :],
                    preferred_element_type=jnp.float32) + b4_ref[:, :]
    o_ref[:, :] = acc

  row_spec = pl.BlockSpec((R, D), lambda i: (i, 0))
  cnt_spec = pl.BlockSpec((R, NW), lambda i: (i, 0))
  w_spec = pl.BlockSpec((D, D), lambda i: (0, 0))
  b_spec = pl.BlockSpec((1, D), lambda i: (0, 0))
  in_specs = [row_spec, row_spec, cnt_spec, row_spec,
              w_spec, w_spec, b_spec]
  args = [p0, p1, cnt_t, h, Wl, Wr, b.reshape(1, D)]
  if fused_final:
    in_specs += [w_spec, b_spec]
    args += [W4, b4.reshape(1, D)]
  return pl.pallas_call(
      tcb,
      grid=(N_NODES // R,),
      in_specs=in_specs,
      out_specs=row_spec,
      out_shape=jax.ShapeDtypeStruct((N_NODES, D), jnp.float32),
  )(*args)


def kernel(x, edge_index, W1l, W1r, b1, W2l, W2r, b2, W3l, W3r, b3, W4, b4):
  ei = edge_index.astype(jnp.int32)
  src = ei[0].reshape(NW, NGRP, GRP, CHUNK)
  dst = ei[1].reshape(NW, NGRP, GRP, CHUNK)
  dstf = ei[1].reshape(NW, EPT)

  agg = _sc_agg()
  cnt = _sc_counts()(dstf)
  cnt_t = cnt.T[:N_NODES]

  p = agg(x, src, dst)
  h1 = _tc_layer(p[0, :N_NODES], p[1, :N_NODES], cnt_t, x, W1l, W1r, b1)
  p2 = agg(h1, src, dst)
  h2 = _tc_layer(p2[0, :N_NODES], p2[1, :N_NODES], cnt_t, h1, W2l, W2r, b2)
  p3 = agg(h2, src, dst)
  return _tc_layer(p3[0, :N_NODES], p3[1, :N_NODES], cnt_t, h2,
                   W3l, W3r, b3, W4, b4)
